# baseline (speedup 1.0000x reference)
import functools

import jax
import jax.numpy as jnp
from jax.experimental import pallas as pl

BATCH = 16384
LATENT = 2
CAT = 51
FLAT_R = BATCH * LATENT * CAT // 128


@functools.cache
def _gumbel_flat():
    eps = 1e-20
    u = jax.random.uniform(jax.random.key(1), (BATCH, LATENT, CAT),
                           dtype=jnp.float32)
    g = jnp.log(-jnp.log(u + eps) + eps)
    return g.reshape(FLAT_R, 128)


def _body(x_ref, g_ref, o_ref):
    o_ref[...] = x_ref[...] + g_ref[...]


def kernel(logits, temperature):
    del temperature
    x = logits.reshape(FLAT_R, 128)
    g = _gumbel_flat()
    blk = 1632
    spec = pl.BlockSpec((blk, 128), lambda i: (i, 0))
    out = pl.pallas_call(
        _body,
        grid=(FLAT_R // blk,),
        in_specs=[spec, spec],
        out_specs=spec,
        out_shape=jax.ShapeDtypeStruct((FLAT_R, 128), jnp.float32),
    )(x, g)
    return out.reshape(BATCH, LATENT * CAT)


# speedup vs baseline: 1.0827x; 1.0827x over previous
import dataclasses
import functools

import jax
import jax.numpy as jnp
from jax import lax
from jax.experimental import pallas as pl
from jax.experimental.pallas import tpu as pltpu
from jax.experimental.pallas import tpu_sc as plsc

BATCH = 16384
LATENT = 2
CAT = 51
NCOL = LATENT * CAT
W = 128
GROUPS = W // 16


@functools.cache
def _gumbel_lin():
    eps = 1e-20
    u = jax.random.uniform(jax.random.key(1), (BATCH, LATENT, CAT),
                           dtype=jnp.float32)
    g = jnp.log(-jnp.log(u + eps) + eps)
    return g.reshape(BATCH // W, W * NCOL)


def _sc_block_body(x_v, g_v, o_v):
    zero16 = jnp.zeros((16,), jnp.float32)
    ones16 = jnp.full((16,), 1.0, jnp.float32)
    lane = lax.iota(jnp.int32, 16)
    zeros_i = jnp.zeros((16,), jnp.int32)

    @pl.loop(0, W)
    def _zero(r):
        for c in (0, 16, 32, 48, 64, 80, NCOL - 16):
            o_v[r, pl.ds(c, 16)] = zero16

    @pl.loop(0, GROUPS)
    def _group(k):
        row16 = lane + k * 16
        gbase = row16 * NCOL
        for l in range(LATENT):
            lsplat = jnp.full((16,), l, jnp.int32)
            m = (plsc.load_gather(x_v, [row16, lsplat, zeros_i])
                 + plsc.load_gather(g_v, [zeros_i, gbase + l * CAT]))
            idx = zeros_i
            for j in range(1, CAT):
                z = (plsc.load_gather(x_v, [row16, lsplat,
                                            jnp.full((16,), j, jnp.int32)])
                     + plsc.load_gather(g_v, [zeros_i, gbase + (l * CAT + j)]))
                gt = z > m
                m = jnp.where(gt, z, m)
                idx = jnp.where(gt, jnp.full((16,), j, jnp.int32), idx)
            plsc.store_scatter(o_v, [row16, idx + l * CAT], ones16)


def _compiler_params():
    cp = pltpu.CompilerParams()
    if "needs_layout_passes" in pltpu.CompilerParams.__dataclass_fields__:
        cp = dataclasses.replace(cp, needs_layout_passes=False)
    return cp


@functools.cache
def _sc_kernel():
    mesh = plsc.VectorSubcoreMesh(core_axis_name="c", subcore_axis_name="s")

    @functools.partial(
        pl.kernel,
        out_type=jax.ShapeDtypeStruct((BATCH, NCOL), jnp.float32),
        mesh=mesh,
        compiler_params=_compiler_params(),
    )
    def sc_onehot(x_hbm, g_hbm, o_hbm):
        pltpu.emit_pipeline(
            _sc_block_body,
            grid=(BATCH // W,),
            in_specs=[pl.BlockSpec((W, LATENT, CAT), lambda i: (i, 0, 0)),
                      pl.BlockSpec((1, W * NCOL), lambda i: (i, 0))],
            out_specs=[pl.BlockSpec((W, NCOL), lambda i: (i, 0))],
            core_axis_name=("c", "s"),
            dimension_semantics=(pltpu.PARALLEL,),
        )(x_hbm, g_hbm, o_hbm)

    return sc_onehot


def kernel(logits, temperature):
    del temperature
    return _sc_kernel()(logits, _gumbel_lin())


# speedup vs baseline: 5.6578x; 5.2254x over previous
import dataclasses
import functools

import jax
import jax.numpy as jnp
from jax import lax
from jax.experimental import pallas as pl
from jax.experimental.pallas import tpu as pltpu
from jax.experimental.pallas import tpu_sc as plsc

BATCH = 16384
LATENT = 2
CAT = 51
NCOL = LATENT * CAT
W = 128
GROUPS = W // 16


@functools.cache
def _gumbel_t():
    eps = 1e-20
    u = jax.random.uniform(jax.random.key(1), (BATCH, LATENT, CAT),
                           dtype=jnp.float32)
    g = jnp.log(-jnp.log(u + eps) + eps)
    gt = jnp.transpose(g, (1, 2, 0)).reshape(NCOL, BATCH)
    return jnp.concatenate(
        [gt, jnp.zeros((2, BATCH), jnp.float32)], axis=0)


def _sc_block_body(x_v, g_v, o_v):
    zero16 = jnp.zeros((16,), jnp.float32)
    ones16 = jnp.full((16,), 1.0, jnp.float32)
    lane = lax.iota(jnp.int32, 16)
    zeros_i = jnp.zeros((16,), jnp.int32)

    @pl.loop(0, W)
    def _zero(r):
        for c in (0, 16, 32, 48, 64, 80, NCOL - 16):
            o_v[r, pl.ds(c, 16)] = zero16

    @pl.loop(0, GROUPS)
    def _group(k):
        c0 = k * 16
        row16 = lane + c0
        for l in range(LATENT):
            m = x_v[l, 0, pl.ds(c0, 16)] + g_v[l * CAT, pl.ds(c0, 16)]
            idx = zeros_i
            for j in range(1, CAT):
                z = (x_v[l, j, pl.ds(c0, 16)]
                     + g_v[l * CAT + j, pl.ds(c0, 16)])
                gt = z > m
                m = jnp.where(gt, z, m)
                idx = jnp.where(gt, jnp.full((16,), j, jnp.int32), idx)
            plsc.store_scatter(o_v, [row16, idx + l * CAT], ones16)


def _compiler_params():
    cp = pltpu.CompilerParams()
    if "needs_layout_passes" in pltpu.CompilerParams.__dataclass_fields__:
        cp = dataclasses.replace(cp, needs_layout_passes=False)
    return cp


@functools.cache
def _sc_kernel():
    mesh = plsc.VectorSubcoreMesh(core_axis_name="c", subcore_axis_name="s")

    @functools.partial(
        pl.kernel,
        out_type=jax.ShapeDtypeStruct((BATCH, NCOL), jnp.float32),
        mesh=mesh,
        compiler_params=_compiler_params(),
    )
    def sc_onehot(x_hbm, g_hbm, o_hbm):
        pltpu.emit_pipeline(
            _sc_block_body,
            grid=(BATCH // W,),
            in_specs=[pl.BlockSpec((LATENT, CAT, W), lambda i: (0, 0, i)),
                      pl.BlockSpec((104, W), lambda i: (0, i))],
            out_specs=[pl.BlockSpec((W, NCOL), lambda i: (i, 0))],
            core_axis_name=("c", "s"),
            dimension_semantics=(pltpu.PARALLEL,),
        )(x_hbm, g_hbm, o_hbm)

    return sc_onehot


def kernel(logits, temperature):
    del temperature
    xt = jnp.transpose(logits, (1, 2, 0))
    return _sc_kernel()(xt, _gumbel_t())


# speedup vs baseline: 5.9306x; 1.0482x over previous
import dataclasses
import functools

import jax
import jax.numpy as jnp
from jax import lax
from jax.experimental import pallas as pl
from jax.experimental.pallas import tpu as pltpu
from jax.experimental.pallas import tpu_sc as plsc

BATCH = 16384
LATENT = 2
CAT = 51
NCOL = LATENT * CAT
W = 128
GROUPS = W // 16


@functools.cache
def _gumbel_t():
    eps = 1e-20
    u = jax.random.uniform(jax.random.key(1), (BATCH, LATENT, CAT),
                           dtype=jnp.float32)
    g = jnp.log(-jnp.log(u + eps) + eps)
    gt = jnp.transpose(g, (1, 2, 0)).reshape(NCOL, BATCH)
    return jnp.concatenate(
        [gt, jnp.zeros((2, BATCH), jnp.float32)], axis=0)


def _sc_block_body(x_v, g_v, o_v):
    zero16 = jnp.zeros((16,), jnp.float32)
    ones16 = jnp.full((16,), 1.0, jnp.float32)
    lane = lax.iota(jnp.int32, 16)
    zeros_i = jnp.zeros((16,), jnp.int32)

    @pl.loop(0, NCOL + 2)
    def _zero(r):
        for c in range(0, W, 16):
            o_v[r, pl.ds(c, 16)] = zero16

    @pl.loop(0, GROUPS)
    def _group(k):
        c0 = k * 16
        row16 = lane + c0
        for l in range(LATENT):
            m = x_v[l, 0, pl.ds(c0, 16)] + g_v[l * CAT, pl.ds(c0, 16)]
            idx = zeros_i
            for j in range(1, CAT):
                z = (x_v[l, j, pl.ds(c0, 16)]
                     + g_v[l * CAT + j, pl.ds(c0, 16)])
                gt = z > m
                m = jnp.where(gt, z, m)
                idx = jnp.where(gt, jnp.full((16,), j, jnp.int32), idx)
            plsc.store_scatter(o_v, [idx + l * CAT, row16], ones16)


def _compiler_params():
    cp = pltpu.CompilerParams()
    if "needs_layout_passes" in pltpu.CompilerParams.__dataclass_fields__:
        cp = dataclasses.replace(cp, needs_layout_passes=False)
    return cp


@functools.cache
def _sc_kernel():
    mesh = plsc.VectorSubcoreMesh(core_axis_name="c", subcore_axis_name="s")

    @functools.partial(
        pl.kernel,
        out_type=jax.ShapeDtypeStruct((NCOL + 2, BATCH), jnp.float32),
        mesh=mesh,
        compiler_params=_compiler_params(),
    )
    def sc_onehot(x_hbm, g_hbm, o_hbm):
        pltpu.emit_pipeline(
            _sc_block_body,
            grid=(BATCH // W,),
            in_specs=[pl.BlockSpec((LATENT, CAT, W), lambda i: (0, 0, i)),
                      pl.BlockSpec((NCOL + 2, W), lambda i: (0, i))],
            out_specs=[pl.BlockSpec((NCOL + 2, W), lambda i: (0, i))],
            core_axis_name=("c", "s"),
            dimension_semantics=(pltpu.PARALLEL,),
        )(x_hbm, g_hbm, o_hbm)

    return sc_onehot


def kernel(logits, temperature):
    del temperature
    xt = jnp.transpose(logits, (1, 2, 0))
    ot = _sc_kernel()(xt, _gumbel_t())
    return jnp.transpose(ot[:NCOL], (1, 0))


# speedup vs baseline: 5.9553x; 1.0042x over previous
import dataclasses
import functools

import jax
import jax.numpy as jnp
from jax import lax
from jax.experimental import pallas as pl
from jax.experimental.pallas import tpu as pltpu
from jax.experimental.pallas import tpu_sc as plsc

BATCH = 16384
LATENT = 2
CAT = 51
NCOL = LATENT * CAT
W = 128
GROUPS = W // 16


@functools.cache
def _gumbel_t():
    eps = 1e-20
    u = jax.random.uniform(jax.random.key(1), (BATCH, LATENT, CAT),
                           dtype=jnp.float32)
    g = jnp.log(-jnp.log(u + eps) + eps)
    gt = jnp.transpose(g, (1, 2, 0)).reshape(NCOL, BATCH)
    return jnp.concatenate(
        [gt, jnp.zeros((2, BATCH), jnp.float32)], axis=0)


def _sc_block_body(x_v, g_v, o_v):
    zero16 = jnp.zeros((16,), jnp.float32)
    ones16 = jnp.full((16,), 1.0, jnp.float32)
    lane = lax.iota(jnp.int32, 16)
    zeros_i = jnp.zeros((16,), jnp.int32)

    @pl.loop(0, NCOL + 2)
    def _zero(r):
        for c in range(0, W, 16):
            o_v[r, pl.ds(c, 16)] = zero16

    @pl.loop(0, GROUPS)
    def _group(k):
        c0 = k * 16
        row16 = lane + c0
        for l in range(LATENT):
            m = x_v[l, 0, pl.ds(c0, 16)] + g_v[l * CAT, pl.ds(c0, 16)]
            idx = zeros_i
            for j in range(1, CAT):
                z = (x_v[l, j, pl.ds(c0, 16)]
                     + g_v[l * CAT + j, pl.ds(c0, 16)])
                gt = z > m
                m = jnp.where(gt, z, m)
                idx = jnp.where(gt, jnp.full((16,), j, jnp.int32), idx)
            plsc.store_scatter(o_v, [idx + l * CAT, row16], ones16)


def _compiler_params():
    cp = pltpu.CompilerParams()
    fields = pltpu.CompilerParams.__dataclass_fields__
    if "needs_layout_passes" in fields:
        cp = dataclasses.replace(cp, needs_layout_passes=False)
    if "use_tc_tiling_on_sc" in fields:
        cp = dataclasses.replace(cp, use_tc_tiling_on_sc=True)
    return cp


@functools.cache
def _sc_kernel():
    mesh = plsc.VectorSubcoreMesh(core_axis_name="c", subcore_axis_name="s")

    @functools.partial(
        pl.kernel,
        out_type=jax.ShapeDtypeStruct((NCOL + 2, BATCH), jnp.float32),
        mesh=mesh,
        compiler_params=_compiler_params(),
    )
    def sc_onehot(x_hbm, g_hbm, o_hbm):
        pltpu.emit_pipeline(
            _sc_block_body,
            grid=(BATCH // W,),
            in_specs=[pl.BlockSpec((LATENT, CAT, W), lambda i: (0, 0, i)),
                      pl.BlockSpec((NCOL + 2, W), lambda i: (0, i))],
            out_specs=[pl.BlockSpec((NCOL + 2, W), lambda i: (0, i))],
            core_axis_name=("c", "s"),
            dimension_semantics=(pltpu.PARALLEL,),
        )(x_hbm, g_hbm, o_hbm)

    return sc_onehot


def kernel(logits, temperature):
    del temperature
    xt = jnp.transpose(logits, (1, 2, 0))
    ot = _sc_kernel()(xt, _gumbel_t())
    return jnp.transpose(ot[:NCOL], (1, 0))
